# baseline (speedup 1.0000x reference)
import functools

import jax
from jax import lax
import jax.numpy as jnp
from jax.experimental import pallas as pl
from jax.experimental.pallas import tpu as pltpu
from jax.experimental.pallas import tpu_sc as plsc


def _presence_rows(s):
    nc = s.shape[0]
    chans = [s[c] for c in range(nc)]
    flags = []
    for c in range(nc):
        is_lab = None
        for j in range(nc):
            if j == c:
                continue
            cmp = (chans[c] > chans[j]) if j < c else (chans[c] >= chans[j])
            is_lab = cmp if is_lab is None else jnp.logical_and(is_lab, cmp)
        flags.append(jnp.max(is_lab.astype(jnp.float32)))
    return flags


def _tc_body(sa_ref, ia_ref, sb_ref, ib_ref, mia_ref, mib_ref, fl_ref):
    b = pl.program_id(0)
    h = pl.program_id(1)

    sa = sa_ref[0]
    sb = sb_ref[0]

    wa = sa[0] + sa[1] + sa[2] + sa[3]
    wb = sb[0] + sb[1] + sb[2] + sb[3]
    mia_ref[0] = wa[None, :, :] * ia_ref[0]
    mib_ref[0] = wb[None, :, :] * ib_ref[0]

    fa = _presence_rows(sa)
    fb = _presence_rows(sb)
    vals = fa + fb
    rows = lax.broadcasted_iota(jnp.int32, (8, 128), 0)
    cur = jnp.zeros((8, 128), jnp.float32)
    for i, v in enumerate(vals):
        cur = jnp.where(rows == i, v, cur)

    @pl.when(jnp.logical_and(b == 0, h == 0))
    def _():
        fl_ref[...] = cur

    @pl.when(jnp.logical_not(jnp.logical_and(b == 0, h == 0)))
    def _():
        fl_ref[...] = jnp.maximum(fl_ref[...], cur)


def _fixup_body(cm_ref, sa_ref, ia_ref, sb_ref, ib_ref,
                ma_ref, mia_ref, mb_ref, mib_ref):
    sa = sa_ref[0]
    sb = sb_ref[0]
    nc = sa.shape[0]
    wa = None
    wb = None
    for c in range(nc):
        cmc = cm_ref[0, c]
        mc_a = sa[c] * cmc
        mc_b = sb[c] * cmc
        ma_ref[0, c] = mc_a
        mb_ref[0, c] = mc_b
        wa = mc_a if wa is None else wa + mc_a
        wb = mc_b if wb is None else wb + mc_b
    mia_ref[0] = wa[None, :, :] * ia_ref[0]
    mib_ref[0] = wb[None, :, :] * ib_ref[0]


_SC_CHUNK = 32768


def _sc_copy_body(total_words, sa_ref, sb_ref, ma_ref, mb_ref,
                  buf0, buf1, in_sem, out_sem):
    nw = 32
    per = total_words // nw
    nchunks = per // _SC_CHUNK
    wid = lax.axis_index("s") * 2 + lax.axis_index("c")
    base = wid * per

    srcs = [sa_ref, sb_ref]
    dsts = [ma_ref, mb_ref]
    tasks = [(arr, k) for arr in range(2) for k in range(nchunks)]
    bufs = [buf0, buf1]

    def in_copy(t, buf):
        arr, k = tasks[t]
        return pltpu.make_async_copy(
            srcs[arr].at[pl.ds(base + k * _SC_CHUNK, _SC_CHUNK)], buf, in_sem)

    def out_copy(t, buf):
        arr, k = tasks[t]
        return pltpu.make_async_copy(
            buf, dsts[arr].at[pl.ds(base + k * _SC_CHUNK, _SC_CHUNK)], out_sem)

    n = len(tasks)
    in_copy(0, bufs[0]).start()
    for t in range(n):
        cur = bufs[t % 2]
        in_copy(t, cur).wait()
        if t >= 1:
            out_copy(t - 1, bufs[(t - 1) % 2]).wait()
        if t + 1 < n:
            in_copy(t + 1, bufs[(t + 1) % 2]).start()
        out_copy(t, cur).start()
    out_copy(n - 1, bufs[(n - 1) % 2]).wait()


def _sc_copy(seg_a, seg_b):
    total = seg_a.size
    flat_a = seg_a.reshape(total)
    flat_b = seg_b.reshape(total)
    f32 = jnp.float32
    mesh = plsc.VectorSubcoreMesh(core_axis_name="c", subcore_axis_name="s")
    fn = pl.kernel(
        functools.partial(_sc_copy_body, total),
        out_type=[jax.ShapeDtypeStruct((total,), f32),
                  jax.ShapeDtypeStruct((total,), f32)],
        mesh=mesh,
        scratch_types=[
            pltpu.VMEM((_SC_CHUNK,), f32),
            pltpu.VMEM((_SC_CHUNK,), f32),
            pltpu.SemaphoreType.DMA,
            pltpu.SemaphoreType.DMA,
        ],
    )
    ma, mb = fn(flat_a, flat_b)
    return ma.reshape(seg_a.shape), mb.reshape(seg_b.shape)


@functools.partial(jax.jit, static_argnames=("bh",))
def _run(img_a, seg_a, img_b, seg_b, bh=256):
    B, C, H, W = img_a.shape
    NC = seg_a.shape[1]
    grid = (B, H // bh)

    seg_spec = pl.BlockSpec((1, NC, bh, W), lambda b, h: (b, 0, h, 0))
    img_spec = pl.BlockSpec((1, C, bh, W), lambda b, h: (b, 0, h, 0))
    fl_spec = pl.BlockSpec((8, 128), lambda b, h: (0, 0))

    f32 = jnp.float32
    mia, mib, flags = pl.pallas_call(
        _tc_body,
        grid=grid,
        in_specs=[seg_spec, img_spec, seg_spec, img_spec],
        out_specs=[img_spec, img_spec, fl_spec],
        out_shape=[
            jax.ShapeDtypeStruct((B, C, H, W), f32),
            jax.ShapeDtypeStruct((B, C, H, W), f32),
            jax.ShapeDtypeStruct((8, 128), f32),
        ],
    )(seg_a, img_a, seg_b, img_b)

    ma, mb = _sc_copy(seg_a, seg_b)

    pa = flags[:4, 0]
    pb = flags[4:8, 0]
    common = pa * pb
    all_common = jnp.all(common > 0.5)

    def fast(_):
        return mia, ma, mib, mb

    def slow(_):
        cm = common.reshape(1, NC)
        cm_spec = pl.BlockSpec(memory_space=pltpu.SMEM)
        o_ma, o_mia, o_mb, o_mib = pl.pallas_call(
            _fixup_body,
            grid=grid,
            in_specs=[cm_spec, seg_spec, img_spec, seg_spec, img_spec],
            out_specs=[seg_spec, img_spec, seg_spec, img_spec],
            out_shape=[
                jax.ShapeDtypeStruct((B, NC, H, W), f32),
                jax.ShapeDtypeStruct((B, C, H, W), f32),
                jax.ShapeDtypeStruct((B, NC, H, W), f32),
                jax.ShapeDtypeStruct((B, C, H, W), f32),
            ],
        )(cm, seg_a, img_a, seg_b, img_b)
        return o_mia, o_ma, o_mib, o_mb

    return lax.cond(all_common, fast, slow, None)


def kernel(img_a, seg_a, img_b, seg_b):
    return _run(img_a, seg_a, img_b, seg_b)


# speedup vs baseline: 2.0899x; 2.0899x over previous
import functools

import jax
from jax import lax
import jax.numpy as jnp
from jax.experimental import pallas as pl
from jax.experimental.pallas import tpu as pltpu
from jax.experimental.pallas import tpu_sc as plsc


def _presence_rows(s):
    nc = s.shape[0]
    chans = [s[c] for c in range(nc)]
    flags = []
    for c in range(nc):
        is_lab = None
        for j in range(nc):
            if j == c:
                continue
            cmp = (chans[c] > chans[j]) if j < c else (chans[c] >= chans[j])
            is_lab = cmp if is_lab is None else jnp.logical_and(is_lab, cmp)
        flags.append(jnp.max(is_lab.astype(jnp.float32)))
    return flags


def _tc_body(sa_ref, ia_ref, sb_ref, ib_ref, mia_ref, mib_ref, fl_ref):
    b = pl.program_id(0)
    h = pl.program_id(1)

    sa = sa_ref[0]
    sb = sb_ref[0]

    wa = sa[0] + sa[1] + sa[2] + sa[3]
    wb = sb[0] + sb[1] + sb[2] + sb[3]
    mia_ref[0] = wa[None, :, :] * ia_ref[0]
    mib_ref[0] = wb[None, :, :] * ib_ref[0]

    fa = _presence_rows(sa)
    fb = _presence_rows(sb)
    vals = fa + fb
    rows = lax.broadcasted_iota(jnp.int32, (8, 128), 0)
    cur = jnp.zeros((8, 128), jnp.float32)
    for i, v in enumerate(vals):
        cur = jnp.where(rows == i, v, cur)

    @pl.when(jnp.logical_and(b == 0, h == 0))
    def _():
        fl_ref[...] = cur

    @pl.when(jnp.logical_not(jnp.logical_and(b == 0, h == 0)))
    def _():
        fl_ref[...] = jnp.maximum(fl_ref[...], cur)


def _fixup_body(cm_ref, sa_ref, ia_ref, sb_ref, ib_ref,
                ma_ref, mia_ref, mb_ref, mib_ref):
    sa = sa_ref[0]
    sb = sb_ref[0]
    nc = sa.shape[0]
    wa = None
    wb = None
    for c in range(nc):
        cmc = cm_ref[0, c]
        mc_a = sa[c] * cmc
        mc_b = sb[c] * cmc
        ma_ref[0, c] = mc_a
        mb_ref[0, c] = mc_b
        wa = mc_a if wa is None else wa + mc_a
        wb = mc_b if wb is None else wb + mc_b
    mia_ref[0] = wa[None, :, :] * ia_ref[0]
    mib_ref[0] = wb[None, :, :] * ib_ref[0]


_SC_ROWS = 64


def _sc_copy_body(dims, sa_ref, sb_ref, ma_ref, mb_ref,
                  buf0, buf1, in_sem, out_sem):
    B, NC, H, W = dims
    nchunks = H // _SC_ROWS
    wid = lax.axis_index("s") * 2 + lax.axis_index("c")
    b_idx = wid // NC
    c_idx = wid % NC

    srcs = [sa_ref, sb_ref]
    dsts = [ma_ref, mb_ref]
    tasks = [(arr, k) for arr in range(2) for k in range(nchunks)]
    bufs = [buf0, buf1]

    def in_copy(t, buf):
        arr, k = tasks[t]
        return pltpu.make_async_copy(
            srcs[arr].at[b_idx, c_idx, pl.ds(k * _SC_ROWS, _SC_ROWS), :],
            buf, in_sem)

    def out_copy(t, buf):
        arr, k = tasks[t]
        return pltpu.make_async_copy(
            buf, dsts[arr].at[b_idx, c_idx, pl.ds(k * _SC_ROWS, _SC_ROWS), :],
            out_sem)

    n = len(tasks)
    in_copy(0, bufs[0]).start()
    for t in range(n):
        cur = bufs[t % 2]
        in_copy(t, cur).wait()
        if t >= 1:
            out_copy(t - 1, bufs[(t - 1) % 2]).wait()
        if t + 1 < n:
            in_copy(t + 1, bufs[(t + 1) % 2]).start()
        out_copy(t, cur).start()
    out_copy(n - 1, bufs[(n - 1) % 2]).wait()


def _sc_copy(seg_a, seg_b):
    B, NC, H, W = seg_a.shape
    f32 = jnp.float32
    mesh = plsc.VectorSubcoreMesh(core_axis_name="c", subcore_axis_name="s")
    fn = pl.kernel(
        functools.partial(_sc_copy_body, (B, NC, H, W)),
        out_type=[jax.ShapeDtypeStruct(seg_a.shape, f32),
                  jax.ShapeDtypeStruct(seg_b.shape, f32)],
        mesh=mesh,
        scratch_types=[
            pltpu.VMEM((_SC_ROWS, W), f32),
            pltpu.VMEM((_SC_ROWS, W), f32),
            pltpu.SemaphoreType.DMA,
            pltpu.SemaphoreType.DMA,
        ],
        compiler_params=pltpu.CompilerParams(use_tc_tiling_on_sc=True),
    )
    return fn(seg_a, seg_b)


@functools.partial(jax.jit, static_argnames=("bh",))
def _run(img_a, seg_a, img_b, seg_b, bh=256):
    B, C, H, W = img_a.shape
    NC = seg_a.shape[1]
    grid = (B, H // bh)

    seg_spec = pl.BlockSpec((1, NC, bh, W), lambda b, h: (b, 0, h, 0))
    img_spec = pl.BlockSpec((1, C, bh, W), lambda b, h: (b, 0, h, 0))
    fl_spec = pl.BlockSpec((8, 128), lambda b, h: (0, 0))

    f32 = jnp.float32
    mia, mib, flags = pl.pallas_call(
        _tc_body,
        grid=grid,
        in_specs=[seg_spec, img_spec, seg_spec, img_spec],
        out_specs=[img_spec, img_spec, fl_spec],
        out_shape=[
            jax.ShapeDtypeStruct((B, C, H, W), f32),
            jax.ShapeDtypeStruct((B, C, H, W), f32),
            jax.ShapeDtypeStruct((8, 128), f32),
        ],
    )(seg_a, img_a, seg_b, img_b)

    ma, mb = _sc_copy(seg_a, seg_b)

    pa = flags[:4, 0]
    pb = flags[4:8, 0]
    common = pa * pb
    all_common = jnp.all(common > 0.5)

    def fast(_):
        return mia, ma, mib, mb

    def slow(_):
        cm = common.reshape(1, NC)
        cm_spec = pl.BlockSpec(memory_space=pltpu.SMEM)
        o_ma, o_mia, o_mb, o_mib = pl.pallas_call(
            _fixup_body,
            grid=grid,
            in_specs=[cm_spec, seg_spec, img_spec, seg_spec, img_spec],
            out_specs=[seg_spec, img_spec, seg_spec, img_spec],
            out_shape=[
                jax.ShapeDtypeStruct((B, NC, H, W), f32),
                jax.ShapeDtypeStruct((B, C, H, W), f32),
                jax.ShapeDtypeStruct((B, NC, H, W), f32),
                jax.ShapeDtypeStruct((B, C, H, W), f32),
            ],
        )(cm, seg_a, img_a, seg_b, img_b)
        return o_mia, o_ma, o_mib, o_mb

    return lax.cond(all_common, fast, slow, None)


def kernel(img_a, seg_a, img_b, seg_b):
    return _run(img_a, seg_a, img_b, seg_b)


# speedup vs baseline: 2.6104x; 1.2491x over previous
import functools

import jax
from jax import lax
import jax.numpy as jnp
from jax.experimental import pallas as pl
from jax.experimental.pallas import tpu as pltpu
from jax.experimental.pallas import tpu_sc as plsc


def _stream_body(sa_ref, ia_ref, sb_ref, ib_ref,
                 ma_ref, mia_ref, mb_ref, mib_ref):
    sa = sa_ref[0]
    sb = sb_ref[0]
    ma_ref[0] = sa
    mb_ref[0] = sb
    wa = sa[0] + sa[1] + sa[2] + sa[3]
    wb = sb[0] + sb[1] + sb[2] + sb[3]
    mia_ref[0] = wa[None, :, :] * ia_ref[0]
    mib_ref[0] = wb[None, :, :] * ib_ref[0]


_SC_SROWS = 8


def _sc_sample_body(dims, sa_ref, sb_ref, fl_ref, buf, out_v, sem):
    B, NC, H, W = dims
    wid = lax.axis_index("s") * 2 + lax.axis_index("c")
    arr = wid % 2
    b_idx = (wid // 2) % B
    band = wid // (2 * B)
    row0 = band * _SC_SROWS

    @pl.when(arr == 0)
    def _():
        cp = pltpu.make_async_copy(
            sa_ref.at[b_idx, :, pl.ds(row0, _SC_SROWS), :], buf, sem)
        cp.start()
        cp.wait()

    @pl.when(arr == 1)
    def _():
        cp = pltpu.make_async_copy(
            sb_ref.at[b_idx, :, pl.ds(row0, _SC_SROWS), :], buf, sem)
        cp.start()
        cp.wait()

    zeros = jnp.zeros((16,), jnp.float32)
    ones = zeros + 1.0

    def row_step(r, accs):
        accs = list(accs)
        for j in range(W // 16):
            s = [buf[c, r, pl.ds(j * 16, 16)] for c in range(NC)]
            for c in range(NC):
                is_lab = None
                for k in range(NC):
                    if k == c:
                        continue
                    cmp = (s[c] > s[k]) if k < c else (s[c] >= s[k])
                    is_lab = cmp if is_lab is None else jnp.logical_and(is_lab, cmp)
                accs[c] = jnp.maximum(accs[c], jnp.where(is_lab, ones, zeros))
        return tuple(accs)

    accs = lax.fori_loop(0, _SC_SROWS, row_step,
                         tuple(zeros for _ in range(NC)))

    lanes = lax.broadcasted_iota(jnp.int32, (16,), 0)
    out = zeros
    for c in range(NC):
        fc = lax.reduce_max(accs[c], (0,))
        out = jnp.where(lanes == c, fc, out)
    out_v[...] = out
    pltpu.make_async_copy(out_v, fl_ref.at[wid], sem).start()
    pltpu.make_async_copy(out_v, fl_ref.at[wid], sem).wait()


def _sc_sample_flags(seg_a, seg_b):
    B, NC, H, W = seg_a.shape
    f32 = jnp.float32
    mesh = plsc.VectorSubcoreMesh(core_axis_name="c", subcore_axis_name="s")
    fn = pl.kernel(
        functools.partial(_sc_sample_body, (B, NC, H, W)),
        out_type=jax.ShapeDtypeStruct((32, 16), f32),
        mesh=mesh,
        scratch_types=[
            pltpu.VMEM((NC, _SC_SROWS, W), f32),
            pltpu.VMEM((16,), f32),
            pltpu.SemaphoreType.DMA,
        ],
        compiler_params=pltpu.CompilerParams(use_tc_tiling_on_sc=True, needs_layout_passes=False),
    )
    flags = fn(seg_a, seg_b)
    pa = jnp.max(flags[0::2, :NC], axis=0)
    pb = jnp.max(flags[1::2, :NC], axis=0)
    return pa, pb


def _presence_rows(s):
    nc = s.shape[0]
    chans = [s[c] for c in range(nc)]
    flags = []
    for c in range(nc):
        is_lab = None
        for j in range(nc):
            if j == c:
                continue
            cmp = (chans[c] > chans[j]) if j < c else (chans[c] >= chans[j])
            is_lab = cmp if is_lab is None else jnp.logical_and(is_lab, cmp)
        flags.append(jnp.max(is_lab.astype(jnp.float32)))
    return flags


def _full_flags_body(sa_ref, sb_ref, fl_ref):
    b = pl.program_id(0)
    h = pl.program_id(1)
    vals = _presence_rows(sa_ref[0]) + _presence_rows(sb_ref[0])
    rows = lax.broadcasted_iota(jnp.int32, (8, 128), 0)
    cur = jnp.zeros((8, 128), jnp.float32)
    for i, v in enumerate(vals):
        cur = jnp.where(rows == i, v, cur)

    @pl.when(jnp.logical_and(b == 0, h == 0))
    def _():
        fl_ref[...] = cur

    @pl.when(jnp.logical_not(jnp.logical_and(b == 0, h == 0)))
    def _():
        fl_ref[...] = jnp.maximum(fl_ref[...], cur)


def _fixup_body(cm_ref, sa_ref, ia_ref, sb_ref, ib_ref,
                ma_ref, mia_ref, mb_ref, mib_ref):
    sa = sa_ref[0]
    sb = sb_ref[0]
    nc = sa.shape[0]
    wa = None
    wb = None
    for c in range(nc):
        cmc = cm_ref[0, c]
        mc_a = sa[c] * cmc
        mc_b = sb[c] * cmc
        ma_ref[0, c] = mc_a
        mb_ref[0, c] = mc_b
        wa = mc_a if wa is None else wa + mc_a
        wb = mc_b if wb is None else wb + mc_b
    mia_ref[0] = wa[None, :, :] * ia_ref[0]
    mib_ref[0] = wb[None, :, :] * ib_ref[0]


@functools.partial(jax.jit, static_argnames=("bh",))
def _run(img_a, seg_a, img_b, seg_b, bh=256):
    B, C, H, W = img_a.shape
    NC = seg_a.shape[1]
    grid = (B, H // bh)

    seg_spec = pl.BlockSpec((1, NC, bh, W), lambda b, h: (b, 0, h, 0))
    img_spec = pl.BlockSpec((1, C, bh, W), lambda b, h: (b, 0, h, 0))
    fl_spec = pl.BlockSpec((8, 128), lambda b, h: (0, 0))
    f32 = jnp.float32

    pa_s, pb_s = _sc_sample_flags(seg_a, seg_b)

    ma, mia, mb, mib = pl.pallas_call(
        _stream_body,
        grid=grid,
        in_specs=[seg_spec, img_spec, seg_spec, img_spec],
        out_specs=[seg_spec, img_spec, seg_spec, img_spec],
        out_shape=[
            jax.ShapeDtypeStruct((B, NC, H, W), f32),
            jax.ShapeDtypeStruct((B, C, H, W), f32),
            jax.ShapeDtypeStruct((B, NC, H, W), f32),
            jax.ShapeDtypeStruct((B, C, H, W), f32),
        ],
    )(seg_a, img_a, seg_b, img_b)

    sample_all_common = jnp.all(pa_s * pb_s > 0.5)

    def fast(_):
        return mia, ma, mib, mb

    def slow(_):
        flags = pl.pallas_call(
            _full_flags_body,
            grid=grid,
            in_specs=[seg_spec, seg_spec],
            out_specs=[fl_spec],
            out_shape=[jax.ShapeDtypeStruct((8, 128), f32)],
        )(seg_a, seg_b)[0]
        common = flags[:4, 0] * flags[4:8, 0]
        cm = common.reshape(1, NC)
        cm_spec = pl.BlockSpec(memory_space=pltpu.SMEM)
        o_ma, o_mia, o_mb, o_mib = pl.pallas_call(
            _fixup_body,
            grid=grid,
            in_specs=[cm_spec, seg_spec, img_spec, seg_spec, img_spec],
            out_specs=[seg_spec, img_spec, seg_spec, img_spec],
            out_shape=[
                jax.ShapeDtypeStruct((B, NC, H, W), f32),
                jax.ShapeDtypeStruct((B, C, H, W), f32),
                jax.ShapeDtypeStruct((B, NC, H, W), f32),
                jax.ShapeDtypeStruct((B, C, H, W), f32),
            ],
        )(cm, seg_a, img_a, seg_b, img_b)
        return o_mia, o_ma, o_mib, o_mb

    return lax.cond(sample_all_common, fast, slow, None)


def kernel(img_a, seg_a, img_b, seg_b):
    return _run(img_a, seg_a, img_b, seg_b)


# speedup vs baseline: 3.1412x; 1.2033x over previous
import functools

import jax
import jax.numpy as jnp
from jax.experimental import pallas as pl
from jax.experimental.pallas import tpu as pltpu


def _presence_rows(s):
    nc = s.shape[0]
    chans = [s[c] for c in range(nc)]
    m = chans[0]
    for c in range(1, nc):
        m = jnp.maximum(m, chans[c])
    eqs = [chans[c] == m for c in range(nc)]
    flags = []
    seen = None
    for c in range(nc):
        is_lab = eqs[c] if seen is None else jnp.logical_and(eqs[c], jnp.logical_not(seen))
        seen = eqs[c] if seen is None else jnp.logical_or(seen, eqs[c])
        flags.append(jnp.max(is_lab.astype(jnp.float32)))
    return flags


def _fused_body(sa_ref, ia_ref, sb_ref, ib_ref,
                ma_ref, mia_ref, mb_ref, mib_ref, fl_ref):
    b = pl.program_id(0)
    h = pl.program_id(1)

    sa = sa_ref[0]
    sb = sb_ref[0]

    ma_ref[0] = sa
    mb_ref[0] = sb
    wa = sa[0] + sa[1] + sa[2] + sa[3]
    wb = sb[0] + sb[1] + sb[2] + sb[3]
    mia_ref[0] = wa[None, :, :] * ia_ref[0]
    mib_ref[0] = wb[None, :, :] * ib_ref[0]

    fa = _presence_rows(sa)
    fb = _presence_rows(sb)
    vals = fa + fb
    rows = jax.lax.broadcasted_iota(jnp.int32, (8, 128), 0)
    cur = jnp.zeros((8, 128), jnp.float32)
    for i, v in enumerate(vals):
        cur = jnp.where(rows == i, v, cur)

    @pl.when(jnp.logical_and(b == 0, h == 0))
    def _():
        fl_ref[...] = cur

    @pl.when(jnp.logical_not(jnp.logical_and(b == 0, h == 0)))
    def _():
        fl_ref[...] = jnp.maximum(fl_ref[...], cur)


def _fixup_body(cm_ref, sa_ref, ia_ref, sb_ref, ib_ref,
                ma_ref, mia_ref, mb_ref, mib_ref):
    sa = sa_ref[0]
    sb = sb_ref[0]
    nc = sa.shape[0]
    wa = None
    wb = None
    for c in range(nc):
        cmc = cm_ref[0, c]
        mc_a = sa[c] * cmc
        mc_b = sb[c] * cmc
        ma_ref[0, c] = mc_a
        mb_ref[0, c] = mc_b
        wa = mc_a if wa is None else wa + mc_a
        wb = mc_b if wb is None else wb + mc_b
    mia_ref[0] = wa[None, :, :] * ia_ref[0]
    mib_ref[0] = wb[None, :, :] * ib_ref[0]


@functools.partial(jax.jit, static_argnames=("bh", "interpret"))
def _run(img_a, seg_a, img_b, seg_b, bh=256, interpret=False):
    B, C, H, W = img_a.shape
    NC = seg_a.shape[1]
    grid = (B, H // bh)

    seg_spec = pl.BlockSpec((1, NC, bh, W), lambda b, h: (b, 0, h, 0))
    img_spec = pl.BlockSpec((1, C, bh, W), lambda b, h: (b, 0, h, 0))
    fl_spec = pl.BlockSpec((8, 128), lambda b, h: (0, 0))

    f32 = jnp.float32
    ma, mia, mb, mib, flags = pl.pallas_call(
        _fused_body,
        grid=grid,
        in_specs=[seg_spec, img_spec, seg_spec, img_spec],
        out_specs=[seg_spec, img_spec, seg_spec, img_spec, fl_spec],
        out_shape=[
            jax.ShapeDtypeStruct((B, NC, H, W), f32),
            jax.ShapeDtypeStruct((B, C, H, W), f32),
            jax.ShapeDtypeStruct((B, NC, H, W), f32),
            jax.ShapeDtypeStruct((B, C, H, W), f32),
            jax.ShapeDtypeStruct((8, 128), f32),
        ],
        interpret=interpret,
    )(seg_a, img_a, seg_b, img_b)

    pa = flags[:4, 0]
    pb = flags[4:8, 0]
    common = pa * pb
    all_common = jnp.all(common > 0.5)

    def fast(_):
        return mia, ma, mib, mb

    def slow(_):
        cm = common.reshape(1, NC)
        cm_spec = pl.BlockSpec(memory_space=pltpu.SMEM)
        o_ma, o_mia, o_mb, o_mib = pl.pallas_call(
            _fixup_body,
            grid=grid,
            in_specs=[cm_spec, seg_spec, img_spec, seg_spec, img_spec],
            out_specs=[seg_spec, img_spec, seg_spec, img_spec],
            out_shape=[
                jax.ShapeDtypeStruct((B, NC, H, W), f32),
                jax.ShapeDtypeStruct((B, C, H, W), f32),
                jax.ShapeDtypeStruct((B, NC, H, W), f32),
                jax.ShapeDtypeStruct((B, C, H, W), f32),
            ],
            interpret=interpret,
        )(cm, seg_a, img_a, seg_b, img_b)
        return o_mia, o_ma, o_mib, o_mb

    return jax.lax.cond(all_common, fast, slow, None)


def kernel(img_a, seg_a, img_b, seg_b):
    return _run(img_a, seg_a, img_b, seg_b, bh=256)


# speedup vs baseline: 3.2121x; 1.0226x over previous
import functools

import jax
import jax.numpy as jnp
from jax.experimental import pallas as pl
from jax.experimental.pallas import tpu as pltpu


def _presence_rows(s):
    nc = s.shape[0]
    chans = [s[c] for c in range(nc)]
    m = chans[0]
    for c in range(1, nc):
        m = jnp.maximum(m, chans[c])
    eqs = [chans[c] == m for c in range(nc)]
    flags = []
    seen = None
    for c in range(nc):
        is_lab = eqs[c] if seen is None else jnp.logical_and(eqs[c], jnp.logical_not(seen))
        seen = eqs[c] if seen is None else jnp.logical_or(seen, eqs[c])
        flags.append(jnp.max(is_lab.astype(jnp.float32)))
    return flags


def _fused_body(sa_ref, ia_ref, sb_ref, ib_ref,
                ma_ref, mia_ref, mb_ref, mib_ref, fl_ref, pred_ref):
    b = pl.program_id(0)
    h = pl.program_id(1)

    sa = sa_ref[0]
    sb = sb_ref[0]

    ma_ref[0] = sa
    mb_ref[0] = sb
    wa = sa[0] + sa[1] + sa[2] + sa[3]
    wb = sb[0] + sb[1] + sb[2] + sb[3]
    mia_ref[0] = wa[None, :, :] * ia_ref[0]
    mib_ref[0] = wb[None, :, :] * ib_ref[0]

    fa = _presence_rows(sa)
    fb = _presence_rows(sb)
    vals = fa + fb
    rows = jax.lax.broadcasted_iota(jnp.int32, (8, 128), 0)
    cur = jnp.zeros((8, 128), jnp.float32)
    for i, v in enumerate(vals):
        cur = jnp.where(rows == i, v, cur)

    @pl.when(jnp.logical_and(b == 0, h == 0))
    def _():
        fl_ref[...] = cur

    @pl.when(jnp.logical_not(jnp.logical_and(b == 0, h == 0)))
    def _():
        fl_ref[...] = jnp.maximum(fl_ref[...], cur)

    nb = pl.num_programs(0)
    nh = pl.num_programs(1)

    @pl.when(jnp.logical_and(b == nb - 1, h == nh - 1))
    def _():
        pred_ref[0] = (jnp.min(fl_ref[...]) > 0.5).astype(jnp.int32)


def _fixup_body(cm_ref, sa_ref, ia_ref, sb_ref, ib_ref,
                ma_ref, mia_ref, mb_ref, mib_ref):
    sa = sa_ref[0]
    sb = sb_ref[0]
    nc = sa.shape[0]
    wa = None
    wb = None
    for c in range(nc):
        cmc = cm_ref[0, c]
        mc_a = sa[c] * cmc
        mc_b = sb[c] * cmc
        ma_ref[0, c] = mc_a
        mb_ref[0, c] = mc_b
        wa = mc_a if wa is None else wa + mc_a
        wb = mc_b if wb is None else wb + mc_b
    mia_ref[0] = wa[None, :, :] * ia_ref[0]
    mib_ref[0] = wb[None, :, :] * ib_ref[0]


@functools.partial(jax.jit, static_argnames=("bh", "interpret"))
def _run(img_a, seg_a, img_b, seg_b, bh=256, interpret=False):
    B, C, H, W = img_a.shape
    NC = seg_a.shape[1]
    grid = (B, H // bh)

    seg_spec = pl.BlockSpec((1, NC, bh, W), lambda b, h: (b, 0, h, 0))
    img_spec = pl.BlockSpec((1, C, bh, W), lambda b, h: (b, 0, h, 0))
    fl_spec = pl.BlockSpec((8, 128), lambda b, h: (0, 0))

    f32 = jnp.float32
    ma, mia, mb, mib, flags, pred = pl.pallas_call(
        _fused_body,
        grid=grid,
        in_specs=[seg_spec, img_spec, seg_spec, img_spec],
        out_specs=[seg_spec, img_spec, seg_spec, img_spec, fl_spec,
                   pl.BlockSpec(memory_space=pltpu.SMEM)],
        out_shape=[
            jax.ShapeDtypeStruct((B, NC, H, W), f32),
            jax.ShapeDtypeStruct((B, C, H, W), f32),
            jax.ShapeDtypeStruct((B, NC, H, W), f32),
            jax.ShapeDtypeStruct((B, C, H, W), f32),
            jax.ShapeDtypeStruct((8, 128), f32),
            jax.ShapeDtypeStruct((1,), jnp.int32),
        ],
        interpret=interpret,
    )(seg_a, img_a, seg_b, img_b)

    def fast(_):
        return mia, ma, mib, mb

    def slow(_):
        common = flags[:4, 0] * flags[4:8, 0]
        cm = common.reshape(1, NC)
        cm_spec = pl.BlockSpec(memory_space=pltpu.SMEM)
        o_ma, o_mia, o_mb, o_mib = pl.pallas_call(
            _fixup_body,
            grid=grid,
            in_specs=[cm_spec, seg_spec, img_spec, seg_spec, img_spec],
            out_specs=[seg_spec, img_spec, seg_spec, img_spec],
            out_shape=[
                jax.ShapeDtypeStruct((B, NC, H, W), f32),
                jax.ShapeDtypeStruct((B, C, H, W), f32),
                jax.ShapeDtypeStruct((B, NC, H, W), f32),
                jax.ShapeDtypeStruct((B, C, H, W), f32),
            ],
            interpret=interpret,
        )(cm, seg_a, img_a, seg_b, img_b)
        return o_mia, o_ma, o_mib, o_mb

    return jax.lax.switch(pred[0], [slow, fast], None)


def kernel(img_a, seg_a, img_b, seg_b):
    return _run(img_a, seg_a, img_b, seg_b, bh=256)
